# baseline (speedup 1.0000x reference)
import functools

import jax
import jax.numpy as jnp
from jax import lax
from jax.experimental import pallas as pl
from jax.experimental.pallas import tpu as pltpu
from jax.experimental.pallas import tpu_sc as plsc

N = 10000
E = 320000
D = 128
DH = D // 2
NC = 2
NS = 16
NW = NC * NS
DCH = 80
DEG_EPT = E // NW
DEG_NCHUNK = DEG_EPT // DCH
CH = 128
EPT = E // NS
NCHUNK = EPT // CH
TAIL = EPT - NCHUNK * CH
NBUF = 4
NPAD = 10240
RPT = NPAD // NS

_mesh = dict(core_axis_name="c", subcore_axis_name="s")


@functools.partial(
    pl.kernel,
    out_type=jax.ShapeDtypeStruct((NC, NPAD), jnp.float32),
    mesh=plsc.VectorSubcoreMesh(**_mesh),
    scratch_types=[
        pltpu.VMEM((DEG_EPT,), jnp.int32),
        pltpu.VMEM((RPT,), jnp.float32),
        pltpu.VMEM((DCH,), jnp.float32),
        pltpu.VMEM_SHARED((NPAD,), jnp.float32),
    ],
    compiler_params=pltpu.CompilerParams(use_tc_tiling_on_sc=False),
)
def _deg_kernel(edges_hbm, out_hbm, colv, zb, onesb, acc):
    cid = lax.axis_index("c")
    sid = lax.axis_index("s")
    wid = sid * NC + cid
    pltpu.sync_copy(edges_hbm.at[1, pl.ds(wid * DEG_EPT, DEG_EPT)], colv)

    z16 = jnp.zeros((16,), jnp.float32)
    o16 = jnp.ones((16,), jnp.float32)

    @pl.loop(0, RPT // 16)
    def _zero(i):
        zb[pl.ds(i * 16, 16)] = z16

    for i in range(DCH // 16):
        onesb[pl.ds(i * 16, 16)] = o16

    pltpu.sync_copy(zb, acc.at[pl.ds(sid * RPT, RPT)])
    plsc.subcore_barrier()

    @pl.loop(0, DEG_NCHUNK)
    def _scat(ci):
        off = pl.multiple_of(ci * DCH, 8)
        pltpu.sync_copy(onesb, acc.at[colv.at[pl.ds(off, DCH)]], add=True)

    plsc.subcore_barrier()
    pltpu.sync_copy(acc.at[pl.ds(sid * RPT, RPT)],
                    out_hbm.at[cid, pl.ds(sid * RPT, RPT)])


@functools.partial(
    pl.kernel,
    out_type=jax.ShapeDtypeStruct((NC, NPAD, DH), jnp.float32),
    mesh=plsc.VectorSubcoreMesh(**_mesh),
    scratch_types=[
        pltpu.VMEM((EPT,), jnp.int32),
        pltpu.VMEM((EPT,), jnp.int32),
        pltpu.VMEM((NBUF, CH, DH), jnp.float32),
        pltpu.VMEM_SHARED((NPAD, DH), jnp.float32),
        pltpu.SemaphoreType.DMA,
        [pltpu.SemaphoreType.DMA] * NBUF,
        [pltpu.SemaphoreType.DMA] * NBUF,
    ],
    compiler_params=pltpu.CompilerParams(use_tc_tiling_on_sc=False),
)
def _seg_kernel(edges_hbm, h_hbm, out_hbm, rowv, colv, rbuf, acc,
                isem, gsems, ssems):
    cid = lax.axis_index("c")
    sid = lax.axis_index("s")
    icopy1 = pltpu.async_copy(edges_hbm.at[0, pl.ds(sid * EPT, EPT)], rowv,
                              isem)
    icopy2 = pltpu.async_copy(edges_hbm.at[1, pl.ds(sid * EPT, EPT)], colv,
                              isem)

    z16 = jnp.zeros((16,), jnp.float32)
    LPR = DH // 16

    @pl.loop(0, CH * LPR)
    def _zero(t):
        rbuf[0, t // LPR, pl.ds((t % LPR) * 16, 16)] = z16

    for k in range(RPT // CH):
        pltpu.sync_copy(rbuf.at[0],
                        acc.at[pl.ds(sid * RPT + k * CH, CH)])
    icopy1.wait()
    icopy2.wait()
    plsc.subcore_barrier()

    def ids(ci, n=CH):
        off = pl.multiple_of(ci * CH, 8)
        return pl.ds(off, n)

    def gather(ci, b):
        return pltpu.async_copy(h_hbm.at[cid].at[rowv.at[ids(ci)]],
                                rbuf.at[b], gsems[b])

    def gather_wait(ci, b):
        pltpu.make_async_copy(h_hbm.at[cid].at[rowv.at[ids(ci)]],
                              rbuf.at[b], gsems[b]).wait()

    def scat(ci, b):
        return pltpu.async_copy(rbuf.at[b], acc.at[colv.at[ids(ci)]],
                                ssems[b], add=True)

    def scat_wait(ci, b):
        pltpu.make_async_copy(rbuf.at[b], acc.at[colv.at[ids(ci)]],
                              ssems[b]).wait()

    for j in range(NBUF - 1):
        gather(j, j)

    @pl.loop(0, NCHUNK, step=NBUF)
    def _body(c):
        for j in range(NBUF):
            ci = c + j
            bn = (j + NBUF - 1) % NBUF
            gather_wait(ci, j)
            scat(ci, j)

            @pl.when(ci + NBUF - 1 < NCHUNK)
            def _():
                @pl.when(ci >= 1)
                def _():
                    scat_wait(ci - 1, bn)

                gather(ci + NBUF - 1, bn)

    for j in range(NBUF):
        scat_wait(NCHUNK - NBUF + j, j)

    tsrc = h_hbm.at[cid].at[rowv.at[pl.ds(NCHUNK * CH, TAIL)]]
    tdst = rbuf.at[0, pl.ds(0, TAIL)]
    pltpu.async_copy(tsrc, tdst, gsems[0]).wait()
    pltpu.async_copy(tdst, acc.at[colv.at[pl.ds(NCHUNK * CH, TAIL)]],
                     ssems[0], add=True).wait()

    plsc.subcore_barrier()
    for k in range(RPT // CH):
        pltpu.sync_copy(acc.at[pl.ds(sid * RPT + k * CH, CH)],
                        out_hbm.at[cid, pl.ds(sid * RPT + k * CH, CH)])


def _prep_body(pdeg_ref, x_ref, w1t_ref, dinv_ref, h_ref):
    deg = pdeg_ref[0, :N, :] + pdeg_ref[1, :N, :] + 1.0
    dinv = lax.rsqrt(deg)
    dinv_ref[...] = dinv
    h = jnp.dot(x_ref[...], w1t_ref[...], preferred_element_type=jnp.float32)
    h = h * dinv
    h_ref[0] = h[:, :DH]
    h_ref[1] = h[:, DH:]


_prep = pl.pallas_call(
    _prep_body,
    out_shape=(
        jax.ShapeDtypeStruct((N, 1), jnp.float32),
        jax.ShapeDtypeStruct((NC, N, DH), jnp.float32),
    ),
)


def _bn_relu_half(u, g_ref, be_ref, lo):
    mu = jnp.mean(u, axis=0, keepdims=True)
    d = u - mu
    var = jnp.mean(d * d, axis=0, keepdims=True)
    y = d * lax.rsqrt(var + 1e-5) * g_ref[:, lo:lo + DH] + \
        be_ref[:, lo:lo + DH]
    return jnp.maximum(y, 0.0)


def _mid_body(p_ref, h_ref, dinv_ref, g_ref, be_ref, w2t_ref, out_ref):
    dinv = dinv_ref[...]
    u0 = (p_ref[0, :N, :] + h_ref[0]) * dinv
    u1 = (p_ref[1, :N, :] + h_ref[1]) * dinv
    y = jnp.concatenate(
        [_bn_relu_half(u0, g_ref, be_ref, 0),
         _bn_relu_half(u1, g_ref, be_ref, DH)], axis=1)
    h2 = jnp.dot(y, w2t_ref[...], preferred_element_type=jnp.float32)
    h2 = h2 * dinv
    out_ref[0] = h2[:, :DH]
    out_ref[1] = h2[:, DH:]


_mid = pl.pallas_call(
    _mid_body,
    out_shape=jax.ShapeDtypeStruct((NC, N, DH), jnp.float32),
)


def _fin_body(p_ref, h_ref, dinv_ref, g_ref, be_ref, out_ref):
    dinv = dinv_ref[...]
    u0 = (p_ref[0, :N, :] + h_ref[0]) * dinv
    u1 = (p_ref[1, :N, :] + h_ref[1]) * dinv
    out_ref[...] = jnp.concatenate(
        [_bn_relu_half(u0, g_ref, be_ref, 0),
         _bn_relu_half(u1, g_ref, be_ref, DH)], axis=1)


_fin = pl.pallas_call(
    _fin_body,
    out_shape=jax.ShapeDtypeStruct((N, D), jnp.float32),
)


def kernel(x, edge_index, W1, b1, W2, b2, g1, be1, g2, be2):
    pdeg = _deg_kernel(edge_index).reshape(NC, NPAD, 1)
    dinv, h1 = _prep(pdeg, x, W1.T)

    p1 = _seg_kernel(edge_index, h1)
    h2 = _mid(p1, h1, dinv, g1.reshape(1, D), be1.reshape(1, D), W2.T)

    p2 = _seg_kernel(edge_index, h2)
    return _fin(p2, h2, dinv, g2.reshape(1, D), be2.reshape(1, D))


# speedup vs baseline: 1.1652x; 1.1652x over previous
import functools

import jax
import jax.numpy as jnp
from jax import lax
from jax.experimental import pallas as pl
from jax.experimental.pallas import tpu as pltpu
from jax.experimental.pallas import tpu_sc as plsc

N = 10000
E = 320000
D = 128
DH = D // 2
NC = 2
NS = 16
NW = NC * NS
DCH = 80
DEG_EPT = E // NW
DEG_NCHUNK = DEG_EPT // DCH
CH = 128
EPT = E // NS
NCHUNK = EPT // CH
TAIL = EPT - NCHUNK * CH
NBUF = 4
NPAD = 10240
RPT = NPAD // NS
HALF = NPAD // 2
REAL_O = N - HALF

_mesh = dict(core_axis_name="c", subcore_axis_name="s")


@functools.partial(
    pl.kernel,
    out_type=jax.ShapeDtypeStruct((NC, NPAD), jnp.float32),
    mesh=plsc.VectorSubcoreMesh(**_mesh),
    scratch_types=[
        pltpu.VMEM((DEG_EPT,), jnp.int32),
        pltpu.VMEM((RPT,), jnp.float32),
        pltpu.VMEM((DCH,), jnp.float32),
        pltpu.VMEM_SHARED((NPAD,), jnp.float32),
    ],
    compiler_params=pltpu.CompilerParams(use_tc_tiling_on_sc=False),
)
def _deg_kernel(edges_hbm, out_hbm, colv, zb, onesb, acc):
    cid = lax.axis_index("c")
    sid = lax.axis_index("s")
    wid = sid * NC + cid
    pltpu.sync_copy(edges_hbm.at[1, pl.ds(wid * DEG_EPT, DEG_EPT)], colv)

    z16 = jnp.zeros((16,), jnp.float32)
    o16 = jnp.ones((16,), jnp.float32)

    @pl.loop(0, RPT // 16)
    def _zero(i):
        zb[pl.ds(i * 16, 16)] = z16

    for i in range(DCH // 16):
        onesb[pl.ds(i * 16, 16)] = o16

    pltpu.sync_copy(zb, acc.at[pl.ds(sid * RPT, RPT)])
    plsc.subcore_barrier()

    @pl.loop(0, DEG_NCHUNK)
    def _scat(ci):
        off = pl.multiple_of(ci * DCH, 8)
        pltpu.sync_copy(onesb, acc.at[colv.at[pl.ds(off, DCH)]], add=True)

    plsc.subcore_barrier()
    pltpu.sync_copy(acc.at[pl.ds(sid * RPT, RPT)],
                    out_hbm.at[cid, pl.ds(sid * RPT, RPT)])


@functools.partial(
    pl.kernel,
    out_type=jax.ShapeDtypeStruct((NC, NPAD, DH), jnp.float32),
    mesh=plsc.VectorSubcoreMesh(**_mesh),
    scratch_types=[
        pltpu.VMEM((EPT,), jnp.int32),
        pltpu.VMEM((EPT,), jnp.int32),
        pltpu.VMEM((NBUF, CH, DH), jnp.float32),
        pltpu.VMEM_SHARED((NPAD, DH), jnp.float32),
        pltpu.SemaphoreType.DMA,
        [pltpu.SemaphoreType.DMA] * NBUF,
        [pltpu.SemaphoreType.DMA] * NBUF,
    ],
    compiler_params=pltpu.CompilerParams(use_tc_tiling_on_sc=False),
)
def _seg_kernel(edges_hbm, h_hbm, out_hbm, rowv, colv, rbuf, acc,
                isem, gsems, ssems):
    cid = lax.axis_index("c")
    sid = lax.axis_index("s")
    icopy1 = pltpu.async_copy(edges_hbm.at[0, pl.ds(sid * EPT, EPT)], rowv,
                              isem)
    icopy2 = pltpu.async_copy(edges_hbm.at[1, pl.ds(sid * EPT, EPT)], colv,
                              isem)

    z16 = jnp.zeros((16,), jnp.float32)
    LPR = DH // 16

    @pl.loop(0, CH * LPR)
    def _zero(t):
        rbuf[0, t // LPR, pl.ds((t % LPR) * 16, 16)] = z16

    for k in range(RPT // CH):
        pltpu.sync_copy(rbuf.at[0],
                        acc.at[pl.ds(sid * RPT + k * CH, CH)])
    icopy1.wait()
    icopy2.wait()
    plsc.subcore_barrier()

    def ids(ci, n=CH):
        off = pl.multiple_of(ci * CH, 8)
        return pl.ds(off, n)

    def xform(ci, n=CH):
        base = pl.multiple_of(ci * CH, 8)
        for ref in (rowv, colv):
            for t in range(n // 16):
                sl = pl.ds(base + t * 16, 16)
                v = ref[sl]
                slot = v * 2 - jnp.where(v >= HALF, 10239, 0)
                ref[sl] = slot

    def gather(ci, b):
        return pltpu.async_copy(h_hbm.at[cid].at[rowv.at[ids(ci)]],
                                rbuf.at[b], gsems[b])

    def gather_wait(ci, b):
        pltpu.make_async_copy(h_hbm.at[cid].at[rowv.at[ids(ci)]],
                              rbuf.at[b], gsems[b]).wait()

    def scat(ci, b):
        return pltpu.async_copy(rbuf.at[b], acc.at[colv.at[ids(ci)]],
                                ssems[b], add=True)

    def scat_wait(ci, b):
        pltpu.make_async_copy(rbuf.at[b], acc.at[colv.at[ids(ci)]],
                              ssems[b]).wait()

    for j in range(NBUF - 1):
        xform(j)
        gather(j, j)

    @pl.loop(0, NCHUNK, step=NBUF)
    def _body(c):
        for j in range(NBUF):
            ci = c + j
            bn = (j + NBUF - 1) % NBUF
            gather_wait(ci, j)
            scat(ci, j)

            @pl.when(ci + NBUF - 1 < NCHUNK)
            def _():
                @pl.when(ci >= 1)
                def _():
                    scat_wait(ci - 1, bn)

                xform(ci + NBUF - 1)
                gather(ci + NBUF - 1, bn)

    for j in range(NBUF):
        scat_wait(NCHUNK - NBUF + j, j)

    xform(NCHUNK, TAIL)
    tsrc = h_hbm.at[cid].at[rowv.at[pl.ds(NCHUNK * CH, TAIL)]]
    tdst = rbuf.at[0, pl.ds(0, TAIL)]
    pltpu.async_copy(tsrc, tdst, gsems[0]).wait()
    pltpu.async_copy(tdst, acc.at[colv.at[pl.ds(NCHUNK * CH, TAIL)]],
                     ssems[0], add=True).wait()

    plsc.subcore_barrier()
    for k in range(RPT // CH):
        pltpu.sync_copy(acc.at[pl.ds(sid * RPT + k * CH, CH)],
                        out_hbm.at[cid, pl.ds(sid * RPT + k * CH, CH)])


def _split_eo(h):
    e = h[:HALF, :]
    o = jnp.concatenate(
        [h[HALF:, :], jnp.zeros((HALF - REAL_O, D), h.dtype)], axis=0)
    return jnp.stack(
        [jnp.concatenate([e[:, :DH], o[:, :DH]], axis=1),
         jnp.concatenate([e[:, DH:], o[:, DH:]], axis=1)])


def _prep_body(pdeg_ref, x_ref, w1t_ref, dinv_ref, h_ref):
    deg = pdeg_ref[0, :N, :] + pdeg_ref[1, :N, :] + 1.0
    dinv = lax.rsqrt(deg)
    dinv_ref[...] = dinv
    h = jnp.dot(x_ref[...], w1t_ref[...], preferred_element_type=jnp.float32)
    h_ref[...] = _split_eo(h * dinv)


_prep = pl.pallas_call(
    _prep_body,
    out_shape=(
        jax.ShapeDtypeStruct((N, 1), jnp.float32),
        jax.ShapeDtypeStruct((NC, HALF, D), jnp.float32),
    ),
)


def _bn_relu_eo(p_ref, h_ref, dinv_ref, g_ref, be_ref):
    d_e = dinv_ref[:HALF, :]
    d_o = jnp.concatenate(
        [dinv_ref[HALF:, :], jnp.ones((HALF - REAL_O, 1), jnp.float32)],
        axis=0)
    ys = []
    for c in range(NC):
        t = p_ref[c] + h_ref[c]
        u_e = t[:, :DH] * d_e
        u_o = t[:, DH:] * d_o
        mu = (jnp.sum(u_e, 0, keepdims=True)
              + jnp.sum(u_o, 0, keepdims=True)) / N
        de = u_e - mu
        do = u_o - mu
        var = (jnp.sum(de * de, 0, keepdims=True)
               + jnp.sum((do * do)[:REAL_O, :], 0, keepdims=True)) / N
        s = lax.rsqrt(var + 1e-5) * g_ref[:, DH * c:DH * (c + 1)]
        b = be_ref[:, DH * c:DH * (c + 1)]
        ys.append((jnp.maximum(de * s + b, 0.0),
                   jnp.maximum(do * s + b, 0.0)))
    y_e = jnp.concatenate([ys[0][0], ys[1][0]], axis=1)
    y_o = jnp.concatenate([ys[0][1], ys[1][1]], axis=1)
    return y_e, y_o, d_e, d_o


def _mid_body(p_ref, h_ref, dinv_ref, g_ref, be_ref, w2t_ref, out_ref):
    y_e, y_o, d_e, d_o = _bn_relu_eo(p_ref, h_ref, dinv_ref, g_ref, be_ref)
    w2t = w2t_ref[...]
    h2_e = jnp.dot(y_e, w2t, preferred_element_type=jnp.float32) * d_e
    h2_o = jnp.dot(y_o, w2t, preferred_element_type=jnp.float32) * d_o
    h2_o = jnp.concatenate(
        [h2_o[:REAL_O, :], jnp.zeros((HALF - REAL_O, D), jnp.float32)],
        axis=0)
    out_ref[0] = jnp.concatenate([h2_e[:, :DH], h2_o[:, :DH]], axis=1)
    out_ref[1] = jnp.concatenate([h2_e[:, DH:], h2_o[:, DH:]], axis=1)


_mid = pl.pallas_call(
    _mid_body,
    out_shape=jax.ShapeDtypeStruct((NC, HALF, D), jnp.float32),
)


def _fin_body(p_ref, h_ref, dinv_ref, g_ref, be_ref, out_ref):
    y_e, y_o, _, _ = _bn_relu_eo(p_ref, h_ref, dinv_ref, g_ref, be_ref)
    out_ref[...] = jnp.concatenate([y_e, y_o[:REAL_O, :]], axis=0)


_fin = pl.pallas_call(
    _fin_body,
    out_shape=jax.ShapeDtypeStruct((N, D), jnp.float32),
)


def kernel(x, edge_index, W1, b1, W2, b2, g1, be1, g2, be2):
    pdeg = _deg_kernel(edge_index).reshape(NC, NPAD, 1)
    dinv, h1 = _prep(pdeg, x, W1.T)

    p1 = _seg_kernel(edge_index, h1.reshape(NC, NPAD, DH))
    h2 = _mid(p1.reshape(NC, HALF, D), h1, dinv, g1.reshape(1, D),
              be1.reshape(1, D), W2.T)

    p2 = _seg_kernel(edge_index, h2.reshape(NC, NPAD, DH))
    return _fin(p2.reshape(NC, HALF, D), h2, dinv, g2.reshape(1, D),
                be2.reshape(1, D))


# speedup vs baseline: 1.2220x; 1.0487x over previous
import functools

import jax
import jax.numpy as jnp
from jax import lax
from jax.experimental import pallas as pl
from jax.experimental.pallas import tpu as pltpu
from jax.experimental.pallas import tpu_sc as plsc

N = 10000
E = 320000
D = 128
DH = D // 2
NC = 2
NS = 16
NW = NC * NS
DCH = 80
DEG_EPT = E // NW
DEG_NCHUNK = DEG_EPT // DCH
CH = 128
EPT = E // NS
NCHUNK = EPT // CH
TAIL = EPT - NCHUNK * CH
NBUF = 4
NPAD = 10240
RPT = NPAD // NS
HALF = NPAD // 2
REAL_O = N - HALF

_mesh = dict(core_axis_name="c", subcore_axis_name="s")


@functools.partial(
    pl.kernel,
    out_type=jax.ShapeDtypeStruct((NC, NPAD), jnp.float32),
    mesh=plsc.VectorSubcoreMesh(**_mesh),
    scratch_types=[
        pltpu.VMEM((DEG_EPT,), jnp.int32),
        pltpu.VMEM((RPT,), jnp.float32),
        pltpu.VMEM((DCH,), jnp.float32),
        pltpu.VMEM_SHARED((NPAD,), jnp.float32),
    ],
    compiler_params=pltpu.CompilerParams(use_tc_tiling_on_sc=False),
)
def _deg_kernel(edges_hbm, out_hbm, colv, zb, onesb, acc):
    cid = lax.axis_index("c")
    sid = lax.axis_index("s")
    wid = sid * NC + cid
    pltpu.sync_copy(edges_hbm.at[1, pl.ds(wid * DEG_EPT, DEG_EPT)], colv)

    z16 = jnp.zeros((16,), jnp.float32)
    o16 = jnp.ones((16,), jnp.float32)

    @pl.loop(0, RPT // 16)
    def _zero(i):
        zb[pl.ds(i * 16, 16)] = z16

    for i in range(DCH // 16):
        onesb[pl.ds(i * 16, 16)] = o16

    pltpu.sync_copy(zb, acc.at[pl.ds(sid * RPT, RPT)])
    plsc.subcore_barrier()

    @pl.loop(0, DEG_NCHUNK)
    def _scat(ci):
        off = pl.multiple_of(ci * DCH, 8)
        pltpu.sync_copy(onesb, acc.at[colv.at[pl.ds(off, DCH)]], add=True)

    plsc.subcore_barrier()
    pltpu.sync_copy(acc.at[pl.ds(sid * RPT, RPT)],
                    out_hbm.at[cid, pl.ds(sid * RPT, RPT)])


@functools.partial(
    pl.kernel,
    out_type=jax.ShapeDtypeStruct((NC, NPAD, DH), jnp.float32),
    mesh=plsc.VectorSubcoreMesh(**_mesh),
    scratch_types=[
        pltpu.VMEM((EPT,), jnp.int32),
        pltpu.VMEM((EPT,), jnp.int32),
        pltpu.VMEM((NBUF, CH, DH), jnp.float32),
        pltpu.VMEM_SHARED((NPAD, DH), jnp.float32),
        pltpu.SemaphoreType.DMA,
        [pltpu.SemaphoreType.DMA] * NBUF,
        [pltpu.SemaphoreType.DMA] * NBUF,
    ],
    compiler_params=pltpu.CompilerParams(use_tc_tiling_on_sc=False),
)
def _seg_kernel(edges_hbm, h_hbm, out_hbm, rowv, colv, rbuf, acc,
                isem, gsems, ssems):
    cid = lax.axis_index("c")
    sid = lax.axis_index("s")
    icopy1 = pltpu.async_copy(edges_hbm.at[0, pl.ds(sid * EPT, EPT)], rowv,
                              isem)
    icopy2 = pltpu.async_copy(edges_hbm.at[1, pl.ds(sid * EPT, EPT)], colv,
                              isem)

    z16 = jnp.zeros((16,), jnp.float32)
    LPR = DH // 16

    @pl.loop(0, CH * LPR)
    def _zero(t):
        rbuf[0, t // LPR, pl.ds((t % LPR) * 16, 16)] = z16

    for k in range(RPT // CH):
        pltpu.sync_copy(rbuf.at[0],
                        acc.at[pl.ds(sid * RPT + k * CH, CH)])
    icopy1.wait()
    icopy2.wait()
    plsc.subcore_barrier()

    def ids(ci, n=CH):
        off = pl.multiple_of(ci * CH, 8)
        return pl.ds(off, n)

    def xform(ci, n=CH):
        base = pl.multiple_of(ci * CH, 8)
        for ref in (rowv, colv):
            for t in range(n // 16):
                sl = pl.ds(base + t * 16, 16)
                v = ref[sl]
                slot = v * 2 - jnp.where(v >= HALF, 10239, 0)
                ref[sl] = slot

    def gather(ci, b):
        return pltpu.async_copy(h_hbm.at[cid].at[rowv.at[ids(ci)]],
                                rbuf.at[b], gsems[b])

    def gather_wait(ci, b):
        pltpu.make_async_copy(h_hbm.at[cid].at[rowv.at[ids(ci)]],
                              rbuf.at[b], gsems[b]).wait()

    def scat(ci, b):
        return pltpu.async_copy(rbuf.at[b], acc.at[colv.at[ids(ci)]],
                                ssems[b], add=True)

    def scat_wait(ci, b):
        pltpu.make_async_copy(rbuf.at[b], acc.at[colv.at[ids(ci)]],
                              ssems[b]).wait()

    for j in range(NBUF - 1):
        xform(j)
        gather(j, j)

    @pl.loop(0, NCHUNK, step=NBUF)
    def _body(c):
        for j in range(NBUF):
            ci = c + j
            bn = (j + NBUF - 1) % NBUF
            gather_wait(ci, j)
            scat(ci, j)

            @pl.when(ci + NBUF - 1 < NCHUNK)
            def _():
                @pl.when(ci >= 1)
                def _():
                    scat_wait(ci - 1, bn)

                xform(ci + NBUF - 1)
                gather(ci + NBUF - 1, bn)

    for j in range(NBUF):
        scat_wait(NCHUNK - NBUF + j, j)

    xform(NCHUNK, TAIL)
    tsrc = h_hbm.at[cid].at[rowv.at[pl.ds(NCHUNK * CH, TAIL)]]
    tdst = rbuf.at[0, pl.ds(0, TAIL)]
    pltpu.async_copy(tsrc, tdst, gsems[0]).wait()
    pltpu.async_copy(tdst, acc.at[colv.at[pl.ds(NCHUNK * CH, TAIL)]],
                     ssems[0], add=True).wait()

    plsc.subcore_barrier()
    for k in range(RPT // CH):
        pltpu.sync_copy(acc.at[pl.ds(sid * RPT + k * CH, CH)],
                        out_hbm.at[cid, pl.ds(sid * RPT + k * CH, CH)])


def _split_eo(h):
    e = h[:HALF, :]
    o = jnp.concatenate(
        [h[HALF:, :], jnp.zeros((HALF - REAL_O, D), h.dtype)], axis=0)
    return jnp.stack(
        [jnp.concatenate([e[:, :DH], o[:, :DH]], axis=1),
         jnp.concatenate([e[:, DH:], o[:, DH:]], axis=1)])


def _mm_body(x_ref, w1t_ref, o_ref):
    o_ref[...] = jnp.dot(x_ref[...], w1t_ref[...],
                         preferred_element_type=jnp.float32)


_mm = pl.pallas_call(
    _mm_body,
    out_shape=jax.ShapeDtypeStruct((N, D), jnp.float32),
)


def _prep_body(pdeg_ref, hm_ref, dinv_ref, h_ref):
    deg = pdeg_ref[0, :] + pdeg_ref[1, :] + 1.0
    dinv = jnp.reshape(lax.rsqrt(deg), (NPAD, 1))[:N, :]
    dinv_ref[...] = dinv
    h_ref[...] = _split_eo(hm_ref[...] * dinv)


_prep = pl.pallas_call(
    _prep_body,
    out_shape=(
        jax.ShapeDtypeStruct((N, 1), jnp.float32),
        jax.ShapeDtypeStruct((NC, HALF, D), jnp.float32),
    ),
)


def _bn_relu_eo(p_ref, h_ref, dinv_ref, g_ref, be_ref):
    d_e = dinv_ref[:HALF, :]
    d_o = jnp.concatenate(
        [dinv_ref[HALF:, :], jnp.ones((HALF - REAL_O, 1), jnp.float32)],
        axis=0)
    ys = []
    for c in range(NC):
        t = p_ref[c] + h_ref[c]
        u_e = t[:, :DH] * d_e
        u_o = t[:, DH:] * d_o
        mu = (jnp.sum(u_e, 0, keepdims=True)
              + jnp.sum(u_o, 0, keepdims=True)) / N
        de = u_e - mu
        do = u_o - mu
        var = (jnp.sum(de * de, 0, keepdims=True)
               + jnp.sum((do * do)[:REAL_O, :], 0, keepdims=True)) / N
        s = lax.rsqrt(var + 1e-5) * g_ref[:, DH * c:DH * (c + 1)]
        b = be_ref[:, DH * c:DH * (c + 1)]
        ys.append((jnp.maximum(de * s + b, 0.0),
                   jnp.maximum(do * s + b, 0.0)))
    y_e = jnp.concatenate([ys[0][0], ys[1][0]], axis=1)
    y_o = jnp.concatenate([ys[0][1], ys[1][1]], axis=1)
    return y_e, y_o, d_e, d_o


def _mid_body(p_ref, h_ref, dinv_ref, g_ref, be_ref, w2t_ref, out_ref):
    y_e, y_o, d_e, d_o = _bn_relu_eo(p_ref, h_ref, dinv_ref, g_ref, be_ref)
    w2t = w2t_ref[...]
    h2_e = jnp.dot(y_e, w2t, preferred_element_type=jnp.float32) * d_e
    h2_o = jnp.dot(y_o, w2t, preferred_element_type=jnp.float32) * d_o
    h2_o = jnp.concatenate(
        [h2_o[:REAL_O, :], jnp.zeros((HALF - REAL_O, D), jnp.float32)],
        axis=0)
    out_ref[0] = jnp.concatenate([h2_e[:, :DH], h2_o[:, :DH]], axis=1)
    out_ref[1] = jnp.concatenate([h2_e[:, DH:], h2_o[:, DH:]], axis=1)


_mid = pl.pallas_call(
    _mid_body,
    out_shape=jax.ShapeDtypeStruct((NC, HALF, D), jnp.float32),
)


def _fin_body(p_ref, h_ref, dinv_ref, g_ref, be_ref, out_ref):
    y_e, y_o, _, _ = _bn_relu_eo(p_ref, h_ref, dinv_ref, g_ref, be_ref)
    out_ref[...] = jnp.concatenate([y_e, y_o[:REAL_O, :]], axis=0)


_fin = pl.pallas_call(
    _fin_body,
    out_shape=jax.ShapeDtypeStruct((N, D), jnp.float32),
)


def kernel(x, edge_index, W1, b1, W2, b2, g1, be1, g2, be2):
    hm = _mm(x, W1.T)
    pdeg = _deg_kernel(edge_index)
    dinv, h1 = _prep(pdeg, hm)

    p1 = _seg_kernel(edge_index, h1.reshape(NC, NPAD, DH))
    h2 = _mid(p1.reshape(NC, HALF, D), h1, dinv, g1.reshape(1, D),
              be1.reshape(1, D), W2.T)

    p2 = _seg_kernel(edge_index, h2.reshape(NC, NPAD, DH))
    return _fin(p2.reshape(NC, HALF, D), h2, dinv, g2.reshape(1, D),
                be2.reshape(1, D))


# speedup vs baseline: 1.3002x; 1.0640x over previous
import functools

import jax
import jax.numpy as jnp
from jax import lax
from jax.experimental import pallas as pl
from jax.experimental.pallas import tpu as pltpu
from jax.experimental.pallas import tpu_sc as plsc

N = 10000
E = 320000
D = 128
DH = D // 2
NC = 2
NS = 16
NW = NC * NS
DCH = 80
DEG_EPT = E // NW
DEG_NCHUNK = DEG_EPT // DCH
CH = 128
EPT = E // NS
NCHUNK = EPT // CH
TAIL = EPT - NCHUNK * CH
NBUF = 6
NPAD = 10240
RPT = NPAD // NS
HALF = NPAD // 2
REAL_O = N - HALF

_mesh = dict(core_axis_name="c", subcore_axis_name="s")


@functools.partial(
    pl.kernel,
    out_type=jax.ShapeDtypeStruct((NC, NPAD), jnp.float32),
    mesh=plsc.VectorSubcoreMesh(**_mesh),
    scratch_types=[
        pltpu.VMEM((DEG_EPT,), jnp.int32),
        pltpu.VMEM((RPT,), jnp.float32),
        pltpu.VMEM((DCH,), jnp.float32),
        pltpu.VMEM_SHARED((NPAD,), jnp.float32),
    ],
    compiler_params=pltpu.CompilerParams(use_tc_tiling_on_sc=False),
)
def _deg_kernel(edges_hbm, out_hbm, colv, zb, onesb, acc):
    cid = lax.axis_index("c")
    sid = lax.axis_index("s")
    wid = sid * NC + cid
    pltpu.sync_copy(edges_hbm.at[1, pl.ds(wid * DEG_EPT, DEG_EPT)], colv)

    z16 = jnp.zeros((16,), jnp.float32)
    o16 = jnp.ones((16,), jnp.float32)

    @pl.loop(0, RPT // 16)
    def _zero(i):
        zb[pl.ds(i * 16, 16)] = z16

    for i in range(DCH // 16):
        onesb[pl.ds(i * 16, 16)] = o16

    pltpu.sync_copy(zb, acc.at[pl.ds(sid * RPT, RPT)])
    plsc.subcore_barrier()

    @pl.loop(0, DEG_NCHUNK)
    def _scat(ci):
        off = pl.multiple_of(ci * DCH, 8)
        pltpu.sync_copy(onesb, acc.at[colv.at[pl.ds(off, DCH)]], add=True)

    plsc.subcore_barrier()
    pltpu.sync_copy(acc.at[pl.ds(sid * RPT, RPT)],
                    out_hbm.at[cid, pl.ds(sid * RPT, RPT)])


@functools.partial(
    pl.kernel,
    out_type=jax.ShapeDtypeStruct((NC, NPAD, DH), jnp.float32),
    mesh=plsc.VectorSubcoreMesh(**_mesh),
    scratch_types=[
        pltpu.VMEM((EPT,), jnp.int32),
        pltpu.VMEM((EPT,), jnp.int32),
        pltpu.VMEM((NBUF, CH, DH), jnp.float32),
        pltpu.VMEM_SHARED((NPAD, DH), jnp.float32),
        pltpu.SemaphoreType.DMA,
        [pltpu.SemaphoreType.DMA] * NBUF,
        [pltpu.SemaphoreType.DMA] * NBUF,
    ],
    compiler_params=pltpu.CompilerParams(use_tc_tiling_on_sc=False),
)
def _seg_kernel(edges_hbm, h_hbm, out_hbm, rowv, colv, rbuf, acc,
                isem, gsems, ssems):
    cid = lax.axis_index("c")
    sid = lax.axis_index("s")
    icopy1 = pltpu.async_copy(edges_hbm.at[0, pl.ds(sid * EPT, EPT)], rowv,
                              isem)
    icopy2 = pltpu.async_copy(edges_hbm.at[1, pl.ds(sid * EPT, EPT)], colv,
                              isem)

    z16 = jnp.zeros((16,), jnp.float32)
    LPR = DH // 16
    ZCH = 128

    @pl.loop(0, ZCH * LPR)
    def _zero(t):
        rbuf[0, t // LPR, pl.ds((t % LPR) * 16, 16)] = z16

    for k in range(RPT // ZCH):
        pltpu.sync_copy(rbuf.at[0, pl.ds(0, ZCH)],
                        acc.at[pl.ds(sid * RPT + k * ZCH, ZCH)])
    icopy1.wait()
    icopy2.wait()
    plsc.subcore_barrier()

    def ids(ci, n=CH):
        off = pl.multiple_of(ci * CH, 8)
        return pl.ds(off, n)

    def xform(ci, n=CH):
        base = pl.multiple_of(ci * CH, 8)
        for ref in (rowv, colv):
            for t in range(n // 16):
                sl = pl.ds(base + t * 16, 16)
                v = ref[sl]
                slot = v * 2 - jnp.where(v >= HALF, 10239, 0)
                ref[sl] = slot

    def gather(ci, b):
        return pltpu.async_copy(h_hbm.at[cid].at[rowv.at[ids(ci)]],
                                rbuf.at[b], gsems[b])

    def gather_wait(ci, b):
        pltpu.make_async_copy(h_hbm.at[cid].at[rowv.at[ids(ci)]],
                              rbuf.at[b], gsems[b]).wait()

    def scat(ci, b):
        return pltpu.async_copy(rbuf.at[b], acc.at[colv.at[ids(ci)]],
                                ssems[b], add=True)

    def scat_wait(ci, b):
        pltpu.make_async_copy(rbuf.at[b], acc.at[colv.at[ids(ci)]],
                              ssems[b]).wait()

    for j in range(NBUF - 1):
        xform(j)
        gather(j, j)

    @pl.loop(0, NCHUNK, step=NBUF)
    def _body(c):
        for j in range(NBUF):
            ci = c + j
            bn = (j + NBUF - 1) % NBUF
            gather_wait(ci, j)
            scat(ci, j)

            @pl.when(ci + NBUF - 1 < NCHUNK)
            def _():
                @pl.when(ci >= 1)
                def _():
                    scat_wait(ci - 1, bn)

                xform(ci + NBUF - 1)
                gather(ci + NBUF - 1, bn)

    for j in range(NBUF):
        scat_wait(NCHUNK - NBUF + j, j)

    if TAIL:
        xform(NCHUNK, TAIL)
        tsrc = h_hbm.at[cid].at[rowv.at[pl.ds(NCHUNK * CH, TAIL)]]
        tdst = rbuf.at[0, pl.ds(0, TAIL)]
        pltpu.async_copy(tsrc, tdst, gsems[0]).wait()
        pltpu.async_copy(tdst, acc.at[colv.at[pl.ds(NCHUNK * CH, TAIL)]],
                         ssems[0], add=True).wait()

    plsc.subcore_barrier()
    for k in range(RPT // ZCH):
        pltpu.sync_copy(acc.at[pl.ds(sid * RPT + k * ZCH, ZCH)],
                        out_hbm.at[cid, pl.ds(sid * RPT + k * ZCH, ZCH)])


def _split_eo(h):
    e = h[:HALF, :]
    o = jnp.concatenate(
        [h[HALF:, :], jnp.zeros((HALF - REAL_O, D), h.dtype)], axis=0)
    return jnp.stack(
        [jnp.concatenate([e[:, :DH], o[:, :DH]], axis=1),
         jnp.concatenate([e[:, DH:], o[:, DH:]], axis=1)])


def _mm_body(x_ref, w1t_ref, o_ref):
    o_ref[...] = jnp.dot(x_ref[...], w1t_ref[...],
                         preferred_element_type=jnp.float32)


_mm = pl.pallas_call(
    _mm_body,
    out_shape=jax.ShapeDtypeStruct((N, D), jnp.float32),
)


def _prep_body(pdeg_ref, hm_ref, dinv_ref, h_ref):
    deg = pdeg_ref[0, :] + pdeg_ref[1, :] + 1.0
    dinv = jnp.reshape(lax.rsqrt(deg), (NPAD, 1))[:N, :]
    dinv_ref[...] = dinv
    h_ref[...] = _split_eo(hm_ref[...] * dinv)


_prep = pl.pallas_call(
    _prep_body,
    out_shape=(
        jax.ShapeDtypeStruct((N, 1), jnp.float32),
        jax.ShapeDtypeStruct((NC, HALF, D), jnp.float32),
    ),
)


def _bn_relu_eo(p_ref, h_ref, dinv_ref, g_ref, be_ref):
    d_e = dinv_ref[:HALF, :]
    d_o = jnp.concatenate(
        [dinv_ref[HALF:, :], jnp.ones((HALF - REAL_O, 1), jnp.float32)],
        axis=0)
    ys = []
    for c in range(NC):
        t = p_ref[c] + h_ref[c]
        u_e = t[:, :DH] * d_e
        u_o = t[:, DH:] * d_o
        mu = (jnp.sum(u_e, 0, keepdims=True)
              + jnp.sum(u_o, 0, keepdims=True)) / N
        de = u_e - mu
        do = u_o - mu
        var = (jnp.sum(de * de, 0, keepdims=True)
               + jnp.sum((do * do)[:REAL_O, :], 0, keepdims=True)) / N
        s = lax.rsqrt(var + 1e-5) * g_ref[:, DH * c:DH * (c + 1)]
        b = be_ref[:, DH * c:DH * (c + 1)]
        ys.append((jnp.maximum(de * s + b, 0.0),
                   jnp.maximum(do * s + b, 0.0)))
    y_e = jnp.concatenate([ys[0][0], ys[1][0]], axis=1)
    y_o = jnp.concatenate([ys[0][1], ys[1][1]], axis=1)
    return y_e, y_o, d_e, d_o


def _mid_body(p_ref, h_ref, dinv_ref, g_ref, be_ref, w2t_ref, out_ref):
    y_e, y_o, d_e, d_o = _bn_relu_eo(p_ref, h_ref, dinv_ref, g_ref, be_ref)
    w2t = w2t_ref[...]
    h2_e = jnp.dot(y_e, w2t, preferred_element_type=jnp.float32) * d_e
    h2_o = jnp.dot(y_o, w2t, preferred_element_type=jnp.float32) * d_o
    h2_o = jnp.concatenate(
        [h2_o[:REAL_O, :], jnp.zeros((HALF - REAL_O, D), jnp.float32)],
        axis=0)
    out_ref[0] = jnp.concatenate([h2_e[:, :DH], h2_o[:, :DH]], axis=1)
    out_ref[1] = jnp.concatenate([h2_e[:, DH:], h2_o[:, DH:]], axis=1)


_mid = pl.pallas_call(
    _mid_body,
    out_shape=jax.ShapeDtypeStruct((NC, HALF, D), jnp.float32),
)


def _fin_body(p_ref, h_ref, dinv_ref, g_ref, be_ref, out_ref):
    y_e, y_o, _, _ = _bn_relu_eo(p_ref, h_ref, dinv_ref, g_ref, be_ref)
    out_ref[...] = jnp.concatenate([y_e, y_o[:REAL_O, :]], axis=0)


_fin = pl.pallas_call(
    _fin_body,
    out_shape=jax.ShapeDtypeStruct((N, D), jnp.float32),
)


def kernel(x, edge_index, W1, b1, W2, b2, g1, be1, g2, be2):
    hm = _mm(x, W1.T)
    pdeg = _deg_kernel(edge_index)
    dinv, h1 = _prep(pdeg, hm)

    p1 = _seg_kernel(edge_index, h1.reshape(NC, NPAD, DH))
    h2 = _mid(p1.reshape(NC, HALF, D), h1, dinv, g1.reshape(1, D),
              be1.reshape(1, D), W2.T)

    p2 = _seg_kernel(edge_index, h2.reshape(NC, NPAD, DH))
    return _fin(p2.reshape(NC, HALF, D), h2, dinv, g2.reshape(1, D),
                be2.reshape(1, D))
